# baseline (speedup 1.0000x reference)
import functools

import jax
import jax.numpy as jnp
from jax import lax
from jax.experimental import pallas as pl
from jax.experimental.pallas import tpu as pltpu
from jax.experimental.pallas import tpu_sc as plsc

_N = 1048576
_T = 1000
_NC = 2
_NS = 16
_NW = _NC * _NS
_PER_W = _N // _NW
_CHUNK = 8192
_NBUF = 2
_L = 16
_SCHED = (4096, 8192, 8192, 8192, 4096)
assert sum(_SCHED) == _PER_W and max(_SCHED) <= _CHUNK


def _popart_body(values_hbm, ids_hbm, tbl_hbm,
                 out_n_hbm, out_u_hbm, tbl_v, *rest):
    ids_bufs = list(rest[0:_NBUF])
    vals_bufs = list(rest[_NBUF:2 * _NBUF])
    on_bufs = list(rest[2 * _NBUF:3 * _NBUF])
    ou_bufs = list(rest[3 * _NBUF:4 * _NBUF])
    sin = list(rest[4 * _NBUF:5 * _NBUF])
    sout = list(rest[5 * _NBUF:6 * _NBUF])
    tsem = rest[6 * _NBUF]

    wid = lax.axis_index("s") * _NC + lax.axis_index("c")
    base = wid * _PER_W

    offs = []
    o = 0
    for sz in _SCHED:
        offs.append(o)
        o += sz
    nch = len(_SCHED)

    def start_in(ci):
        bi = ci % _NBUF
        off = base + offs[ci]
        sz = _SCHED[ci]
        h1 = pltpu.async_copy(ids_hbm.at[pl.ds(off, sz)],
                              ids_bufs[bi].at[pl.ds(0, sz)], sin[bi])
        h2 = pltpu.async_copy(values_hbm.at[pl.ds(off, sz)],
                              vals_bufs[bi].at[pl.ds(0, sz)], sin[bi])
        return (h1, h2)

    in_h = [None] * nch
    out_h = [None] * nch
    in_h[0] = start_in(0)
    pltpu.async_copy(tbl_hbm, tbl_v, tsem).wait()

    for ci in range(nch):
        bi = ci % _NBUF
        if ci + 1 < nch:
            in_h[ci + 1] = start_in(ci + 1)
        in_h[ci][0].wait()
        in_h[ci][1].wait()
        if ci >= _NBUF:
            for h in out_h[ci - _NBUF]:
                h.wait()

        iv, vv = ids_bufs[bi], vals_bufs[bi]
        onv, ouv = on_bufs[bi], ou_bufs[bi]

        off = base + offs[ci]
        sz = _SCHED[ci]
        half = sz // 2
        hs = []
        for hi in range(2):
            lo = hi * (half // _L)

            @plsc.parallel_loop(lo, lo + half // _L, unroll=4)
            def vec_body(j):
                sl = pl.ds(j * _L, _L)
                tid = iv[sl]
                xv = vv[sl]
                wv = plsc.load_gather(tbl_v, [tid])
                bv = plsc.load_gather(tbl_v, [tid + 1024])
                sv = plsc.load_gather(tbl_v, [tid + 2048])
                mv = plsc.load_gather(tbl_v, [tid + 3072])
                nv = wv * xv + bv
                onv[sl] = nv
                ouv[sl] = sv * nv + mv

            hs.append(pltpu.async_copy(
                on_bufs[bi].at[pl.ds(hi * half, half)],
                out_n_hbm.at[pl.ds(off + hi * half, half)], sout[bi]))
            hs.append(pltpu.async_copy(
                ou_bufs[bi].at[pl.ds(hi * half, half)],
                out_u_hbm.at[pl.ds(off + hi * half, half)], sout[bi]))
        out_h[ci] = tuple(hs)

    for ci in range(max(0, nch - _NBUF), nch):
        for h in out_h[ci]:
            h.wait()


@jax.jit
def kernel(values, task_ids, w, b, sigma, mu):
    mesh = plsc.VectorSubcoreMesh(core_axis_name="c", subcore_axis_name="s")
    f = pl.kernel(
        _popart_body,
        mesh=mesh,
        out_type=[
            jax.ShapeDtypeStruct((_N,), jnp.float32),
            jax.ShapeDtypeStruct((_N,), jnp.float32),
        ],
        scratch_types=(
            [pltpu.VMEM((4096,), jnp.float32)]
            + [pltpu.VMEM((_CHUNK,), jnp.int32)] * _NBUF
            + [pltpu.VMEM((_CHUNK,), jnp.float32)] * (3 * _NBUF)
            + [pltpu.SemaphoreType.DMA] * (2 * _NBUF + 1)
        ),
        compiler_params=pltpu.CompilerParams(needs_layout_passes=False),
    )
    pad = (0, 1024 - _T)
    tbl = jnp.concatenate(
        [jnp.pad(w, pad), jnp.pad(b, pad), jnp.pad(sigma, pad), jnp.pad(mu, pad)])
    out_n, out_u = f(values, task_ids, tbl)
    return (out_n, out_u)


# speedup vs baseline: 1.0258x; 1.0258x over previous
import functools

import jax
import jax.numpy as jnp
from jax import lax
from jax.experimental import pallas as pl
from jax.experimental.pallas import tpu as pltpu
from jax.experimental.pallas import tpu_sc as plsc

_N = 1048576
_T = 1000
_NC = 2
_NS = 16
_NW = _NC * _NS
_PER_W = _N // _NW
_CHUNK = 8192
_NBUF = 2
_L = 16
_SCHED = (4096, 8192, 8192, 8192, 4096)
assert sum(_SCHED) == _PER_W and max(_SCHED) <= _CHUNK


def _popart_body(values_hbm, ids_hbm, tbl_hbm,
                 out_n_hbm, out_u_hbm, tbl_v, *rest):
    ids_bufs = list(rest[0:_NBUF])
    vals_bufs = list(rest[_NBUF:2 * _NBUF])
    on_bufs = list(rest[2 * _NBUF:3 * _NBUF])
    ou_bufs = list(rest[3 * _NBUF:4 * _NBUF])
    sin = list(rest[4 * _NBUF:5 * _NBUF])
    sout = list(rest[5 * _NBUF:6 * _NBUF])
    tsem = rest[6 * _NBUF]

    wid = lax.axis_index("s") * _NC + lax.axis_index("c")
    base = wid * _PER_W

    offs = []
    o = 0
    for sz in _SCHED:
        offs.append(o)
        o += sz
    nch = len(_SCHED)

    def start_in(ci):
        bi = ci % _NBUF
        off = base + offs[ci]
        sz = _SCHED[ci]
        h1 = pltpu.async_copy(ids_hbm.at[pl.ds(off, sz)],
                              ids_bufs[bi].at[pl.ds(0, sz)], sin[bi])
        h2 = pltpu.async_copy(values_hbm.at[pl.ds(off, sz)],
                              vals_bufs[bi].at[pl.ds(0, sz)], sin[bi])
        return (h1, h2)

    in_h = [None] * nch
    out_h = [None] * nch
    in_h[0] = start_in(0)
    pltpu.async_copy(tbl_hbm, tbl_v, tsem).wait()

    for ci in range(nch):
        bi = ci % _NBUF
        if ci + 1 < nch:
            in_h[ci + 1] = start_in(ci + 1)
        in_h[ci][0].wait()
        in_h[ci][1].wait()
        if ci >= _NBUF:
            out_h[ci - _NBUF][0].wait()
            out_h[ci - _NBUF][1].wait()

        iv, vv = ids_bufs[bi], vals_bufs[bi]
        onv, ouv = on_bufs[bi], ou_bufs[bi]

        @plsc.parallel_loop(0, _SCHED[ci] // _L, unroll=4)
        def vec_body(j):
            sl = pl.ds(j * _L, _L)
            tid = iv[sl]
            xv = vv[sl]
            wv = plsc.load_gather(tbl_v, [tid])
            bv = plsc.load_gather(tbl_v, [tid + 1024])
            sv = plsc.load_gather(tbl_v, [tid + 2048])
            mv = plsc.load_gather(tbl_v, [tid + 3072])
            nv = wv * xv + bv
            onv[sl] = nv
            ouv[sl] = sv * nv + mv

        off = base + offs[ci]
        sz = _SCHED[ci]
        out_h[ci] = (
            pltpu.async_copy(on_bufs[bi].at[pl.ds(0, sz)],
                             out_n_hbm.at[pl.ds(off, sz)], sout[bi]),
            pltpu.async_copy(ou_bufs[bi].at[pl.ds(0, sz)],
                             out_u_hbm.at[pl.ds(off, sz)], sout[bi]),
        )

    for ci in range(max(0, nch - _NBUF), nch):
        out_h[ci][0].wait()
        out_h[ci][1].wait()


@jax.jit
def kernel(values, task_ids, w, b, sigma, mu):
    mesh = plsc.VectorSubcoreMesh(core_axis_name="c", subcore_axis_name="s")
    f = pl.kernel(
        _popart_body,
        mesh=mesh,
        out_type=[
            jax.ShapeDtypeStruct((_N,), jnp.float32),
            jax.ShapeDtypeStruct((_N,), jnp.float32),
        ],
        scratch_types=(
            [pltpu.VMEM((4096,), jnp.float32)]
            + [pltpu.VMEM((_CHUNK,), jnp.int32)] * _NBUF
            + [pltpu.VMEM((_CHUNK,), jnp.float32)] * (3 * _NBUF)
            + [pltpu.SemaphoreType.DMA] * (2 * _NBUF + 1)
        ),
        compiler_params=pltpu.CompilerParams(needs_layout_passes=False),
    )
    pad = (0, 1024 - _T)
    tbl = jnp.concatenate(
        [jnp.pad(w, pad), jnp.pad(b, pad), jnp.pad(sigma, pad), jnp.pad(mu, pad)])
    out_n, out_u = f(values, task_ids, tbl)
    return (out_n, out_u)
